# baseline (speedup 1.0000x reference)
import jax
import jax.numpy as jnp
from jax.experimental import pallas as pl
from jax.experimental.pallas import tpu as pltpu

_PREC = jax.lax.Precision.HIGHEST


def _dot(a, b, dims):
    return jax.lax.dot_general(a, b, (dims, ((), ())), precision=_PREC,
                               preferred_element_type=jnp.float32)


def _mm(a, b):
    return _dot(a, b, ((1,), (0,)))


def _cheb_kernel(x0_ref, l_ref, w1_ref, b1_ref, out_ref, s_ref):
    k = pl.program_id(0)
    kk1, g1 = w1_ref.shape[0], w1_ref.shape[1]
    b, n = x0_ref.shape

    @pl.when(k == 0)
    def _():
        lmat = l_ref[...]
        t_pp = x0_ref[...]
        t_p = _mm(t_pp, lmat)
        h1 = w1_ref[0] * t_pp[None] + w1_ref[1] * t_p[None]
        for i in range(2, kk1):
            t_new = 2.0 * _mm(t_p, lmat) - t_pp
            h1 = h1 + w1_ref[i] * t_new[None]
            t_pp, t_p = t_p, t_new
        a0 = jnp.maximum(h1 + b1_ref[...], 0.0).reshape(g1 * b, n)
        s_ref[0] = a0
        out_ref[0] = a0

    @pl.when(k == 1)
    def _():
        a1 = _mm(s_ref[0], l_ref[...])
        s_ref[1] = a1
        out_ref[0] = a1

    @pl.when(k >= 2)
    def _():
        a_new = 2.0 * _mm(s_ref[(k + 1) % 2], l_ref[...]) - s_ref[k % 2]
        s_ref[k % 2] = a_new
        out_ref[0] = a_new


def _proj_kernel(a_ref, w2_ref, b2_ref, o_ref):
    r = _dot(a_ref[...], w2_ref[...], ((0,), (0,)))
    o_ref[...] = jnp.maximum(r + b2_ref[...], 0.0)


def _fc_kernel(h_ref, w_ref, fb_ref, v_ref, vb_ref, o_ref, acc_ref):
    i = pl.program_id(0)
    p = _mm(h_ref[...], w_ref[...])

    @pl.when(i == 0)
    def _():
        acc_ref[...] = p

    @pl.when(i > 0)
    def _():
        acc_ref[...] = acc_ref[...] + p

    @pl.when(i == pl.num_programs(0) - 1)
    def _():
        h = jnp.maximum(acc_ref[...] + fb_ref[...], 0.0)
        logits = _mm(h, v_ref[...]) + vb_ref[...]
        s = logits - jnp.max(logits, axis=1, keepdims=True)
        o_ref[...] = s - jnp.log(jnp.sum(jnp.exp(s), axis=1, keepdims=True))


def kernel(x, L, W1, b1, W2, b2, fc1_W, fc1_b, fc2_W, fc2_b):
    bsz, n, _ = x.shape
    k1, _, g1 = W1.shape
    k2, _, g2 = W2.shape
    d = fc1_W.shape[1]
    c = fc2_W.shape[1]

    x0 = x[:, :, 0]
    w1b = W1[:, 0, :, None, None]
    b1b = b1[:, None, None]

    astack = pl.pallas_call(
        _cheb_kernel,
        grid=(k2,),
        in_specs=[
            pl.BlockSpec((bsz, n), lambda k: (0, 0)),
            pl.BlockSpec((n, n), lambda k: (0, 0)),
            pl.BlockSpec((k1, g1, 1, 1), lambda k: (0, 0, 0, 0)),
            pl.BlockSpec((g1, 1, 1), lambda k: (0, 0, 0)),
        ],
        out_specs=pl.BlockSpec((1, g1 * bsz, n), lambda k: (k, 0, 0)),
        out_shape=jax.ShapeDtypeStruct((k2, g1 * bsz, n), jnp.float32),
        scratch_shapes=[pltpu.VMEM((2, g1 * bsz, n), jnp.float32)],
    )(x0, L, w1b, b1b)

    a2 = astack.reshape(k2 * g1, bsz * n)
    w2r = W2.reshape(k2 * g1, g2)
    nchunk = 8
    ch = (bsz * n) // nchunk
    h2f = pl.pallas_call(
        _proj_kernel,
        grid=(nchunk,),
        in_specs=[
            pl.BlockSpec((k2 * g1, ch), lambda i: (0, i)),
            pl.BlockSpec((k2 * g1, g2), lambda i: (0, 0)),
            pl.BlockSpec((1, g2), lambda i: (0, 0)),
        ],
        out_specs=pl.BlockSpec((ch, g2), lambda i: (i, 0)),
        out_shape=jax.ShapeDtypeStruct((bsz * n, g2), jnp.float32),
    )(a2, w2r, b2[None, :])

    h2 = h2f.reshape(bsz, n * g2)
    nblk = 16
    chw = (n * g2) // nblk
    out = pl.pallas_call(
        _fc_kernel,
        grid=(nblk,),
        in_specs=[
            pl.BlockSpec((bsz, chw), lambda i: (0, i)),
            pl.BlockSpec((chw, d), lambda i: (i, 0)),
            pl.BlockSpec((1, d), lambda i: (0, 0)),
            pl.BlockSpec((d, c), lambda i: (0, 0)),
            pl.BlockSpec((1, c), lambda i: (0, 0)),
        ],
        out_specs=pl.BlockSpec((bsz, c), lambda i: (0, 0)),
        out_shape=jax.ShapeDtypeStruct((bsz, c), jnp.float32),
        scratch_shapes=[pltpu.VMEM((bsz, d), jnp.float32)],
    )(h2, fc1_W, fc1_b[None, :], fc2_W, fc2_b[None, :])
    return out


# speedup vs baseline: 3.0161x; 3.0161x over previous
import jax
import jax.numpy as jnp
from jax.experimental import pallas as pl
from jax.experimental.pallas import tpu as pltpu

_PREC = jax.lax.Precision.DEFAULT


def _dot(a, b, dims):
    return jax.lax.dot_general(a, b, (dims, ((), ())), precision=_PREC,
                               preferred_element_type=jnp.float32)


def _mm(a, b):
    return _dot(a, b, ((1,), (0,)))


def _cheb_kernel(x0_ref, l_ref, w1_ref, b1_ref, out_ref, s_ref):
    k = pl.program_id(0)
    kk1, g1 = w1_ref.shape[0], w1_ref.shape[1]
    b, n = x0_ref.shape

    @pl.when(k == 0)
    def _():
        lmat = l_ref[...]
        t_pp = x0_ref[...]
        t_p = _mm(t_pp, lmat)
        h1 = w1_ref[0] * t_pp[None] + w1_ref[1] * t_p[None]
        for i in range(2, kk1):
            t_new = 2.0 * _mm(t_p, lmat) - t_pp
            h1 = h1 + w1_ref[i] * t_new[None]
            t_pp, t_p = t_p, t_new
        a0 = jnp.maximum(h1 + b1_ref[...], 0.0).reshape(g1 * b, n)
        s_ref[0] = a0
        out_ref[0] = a0

    @pl.when(k == 1)
    def _():
        a1 = _mm(s_ref[0], l_ref[...])
        s_ref[1] = a1
        out_ref[0] = a1

    @pl.when(k >= 2)
    def _():
        a_new = 2.0 * _mm(s_ref[(k + 1) % 2], l_ref[...]) - s_ref[k % 2]
        s_ref[k % 2] = a_new
        out_ref[0] = a_new


def _proj_kernel(a_ref, w2_ref, b2_ref, o_ref):
    r = _dot(a_ref[...], w2_ref[...], ((0,), (0,)))
    o_ref[...] = jnp.maximum(r + b2_ref[...], 0.0)


def _fc_kernel(h_ref, w_ref, fb_ref, v_ref, vb_ref, o_ref, acc_ref):
    i = pl.program_id(0)
    p = _mm(h_ref[...], w_ref[...])

    @pl.when(i == 0)
    def _():
        acc_ref[...] = p

    @pl.when(i > 0)
    def _():
        acc_ref[...] = acc_ref[...] + p

    @pl.when(i == pl.num_programs(0) - 1)
    def _():
        h = jnp.maximum(acc_ref[...] + fb_ref[...], 0.0)
        logits = _mm(h, v_ref[...]) + vb_ref[...]
        s = logits - jnp.max(logits, axis=1, keepdims=True)
        o_ref[...] = s - jnp.log(jnp.sum(jnp.exp(s), axis=1, keepdims=True))


def kernel(x, L, W1, b1, W2, b2, fc1_W, fc1_b, fc2_W, fc2_b):
    bsz, n, _ = x.shape
    k1, _, g1 = W1.shape
    k2, _, g2 = W2.shape
    d = fc1_W.shape[1]
    c = fc2_W.shape[1]

    x0 = x[:, :, 0]
    w1b = W1[:, 0, :, None, None]
    b1b = b1[:, None, None]

    astack = pl.pallas_call(
        _cheb_kernel,
        grid=(k2,),
        in_specs=[
            pl.BlockSpec((bsz, n), lambda k: (0, 0)),
            pl.BlockSpec((n, n), lambda k: (0, 0)),
            pl.BlockSpec((k1, g1, 1, 1), lambda k: (0, 0, 0, 0)),
            pl.BlockSpec((g1, 1, 1), lambda k: (0, 0, 0)),
        ],
        out_specs=pl.BlockSpec((1, g1 * bsz, n), lambda k: (k, 0, 0)),
        out_shape=jax.ShapeDtypeStruct((k2, g1 * bsz, n), jnp.float32),
        scratch_shapes=[pltpu.VMEM((2, g1 * bsz, n), jnp.float32)],
    )(x0, L, w1b, b1b)

    a2 = astack.reshape(k2 * g1, bsz * n)
    w2r = W2.reshape(k2 * g1, g2)
    nchunk = 8
    ch = (bsz * n) // nchunk
    h2f = pl.pallas_call(
        _proj_kernel,
        grid=(nchunk,),
        in_specs=[
            pl.BlockSpec((k2 * g1, ch), lambda i: (0, i)),
            pl.BlockSpec((k2 * g1, g2), lambda i: (0, 0)),
            pl.BlockSpec((1, g2), lambda i: (0, 0)),
        ],
        out_specs=pl.BlockSpec((ch, g2), lambda i: (i, 0)),
        out_shape=jax.ShapeDtypeStruct((bsz * n, g2), jnp.float32),
    )(a2, w2r, b2[None, :])

    h2 = h2f.reshape(bsz, n * g2)
    nblk = 16
    chw = (n * g2) // nblk
    out = pl.pallas_call(
        _fc_kernel,
        grid=(nblk,),
        in_specs=[
            pl.BlockSpec((bsz, chw), lambda i: (0, i)),
            pl.BlockSpec((chw, d), lambda i: (i, 0)),
            pl.BlockSpec((1, d), lambda i: (0, 0)),
            pl.BlockSpec((d, c), lambda i: (0, 0)),
            pl.BlockSpec((1, c), lambda i: (0, 0)),
        ],
        out_specs=pl.BlockSpec((bsz, c), lambda i: (0, 0)),
        out_shape=jax.ShapeDtypeStruct((bsz, c), jnp.float32),
        scratch_shapes=[pltpu.VMEM((bsz, d), jnp.float32)],
    )(h2, fc1_W, fc1_b[None, :], fc2_W, fc2_b[None, :])
    return out


# speedup vs baseline: 3.2613x; 1.0813x over previous
import jax
import jax.numpy as jnp
from jax.experimental import pallas as pl
from jax.experimental.pallas import tpu as pltpu

_PREC = jax.lax.Precision.DEFAULT


def _dot(a, b, dims):
    return jax.lax.dot_general(a, b, (dims, ((), ())), precision=_PREC,
                               preferred_element_type=jnp.float32)


def _mm(a, b):
    return _dot(a, b, ((1,), (0,)))


def _cheb_kernel(x0_ref, l_ref, w1_ref, b1_ref, w2_ref, b2_ref, out_ref,
                 s_ref):
    k = pl.program_id(0)
    nk = pl.num_programs(0)
    kk1, g1 = w1_ref.shape[0], w1_ref.shape[1]
    b, n = x0_ref.shape

    def contrib(a):
        a2d = a.reshape(g1, b, n).reshape(g1, b * n)
        return _dot(w2_ref[0], a2d, ((0,), (0,)))

    @pl.when(k == 0)
    def _():
        lmat = l_ref[...]
        t_pp = x0_ref[...]
        t_p = _mm(t_pp, lmat)
        h1 = w1_ref[0] * t_pp[None] + w1_ref[1] * t_p[None]
        for i in range(2, kk1):
            t_new = 2.0 * _mm(t_p, lmat) - t_pp
            h1 = h1 + w1_ref[i] * t_new[None]
            t_pp, t_p = t_p, t_new
        a0 = jnp.maximum(h1 + b1_ref[...], 0.0).reshape(g1 * b, n)
        s_ref[0] = a0
        out_ref[...] = contrib(a0)

    @pl.when(k == 1)
    def _():
        a1 = _mm(s_ref[0], l_ref[...])
        s_ref[1] = a1
        out_ref[...] = out_ref[...] + contrib(a1)

    @pl.when(k >= 2)
    def _():
        a_new = 2.0 * _mm(s_ref[(k + 1) % 2], l_ref[...]) - s_ref[k % 2]
        s_ref[k % 2] = a_new
        tot = out_ref[...] + contrib(a_new)

        @pl.when(k < nk - 1)
        def _():
            out_ref[...] = tot

        @pl.when(k == nk - 1)
        def _():
            out_ref[...] = jnp.maximum(tot + b2_ref[...], 0.0)


def _fc_kernel(h_ref, w_ref, fb_ref, v_ref, vb_ref, o_ref, acc_ref):
    i = pl.program_id(0)
    p = _mm(h_ref[...], w_ref[...])

    @pl.when(i == 0)
    def _():
        acc_ref[...] = p

    @pl.when(i > 0)
    def _():
        acc_ref[...] = acc_ref[...] + p

    @pl.when(i == pl.num_programs(0) - 1)
    def _():
        h = jnp.maximum(acc_ref[...] + fb_ref[...], 0.0)
        logits = _mm(h, v_ref[...]) + vb_ref[...]
        s = logits - jnp.max(logits, axis=1, keepdims=True)
        o_ref[...] = s - jnp.log(jnp.sum(jnp.exp(s), axis=1, keepdims=True))


def kernel(x, L, W1, b1, W2, b2, fc1_W, fc1_b, fc2_W, fc2_b):
    bsz, n, _ = x.shape
    k1, _, g1 = W1.shape
    k2, _, g2 = W2.shape
    d = fc1_W.shape[1]
    c = fc2_W.shape[1]

    x0 = x[:, :, 0]
    w1b = W1[:, 0, :, None, None]
    b1b = b1[:, None, None]

    h2g = pl.pallas_call(
        _cheb_kernel,
        grid=(k2,),
        in_specs=[
            pl.BlockSpec((bsz, n), lambda k: (0, 0)),
            pl.BlockSpec((n, n), lambda k: (0, 0)),
            pl.BlockSpec((k1, g1, 1, 1), lambda k: (0, 0, 0, 0)),
            pl.BlockSpec((g1, 1, 1), lambda k: (0, 0, 0)),
            pl.BlockSpec((1, g1, g2), lambda k: (k, 0, 0)),
            pl.BlockSpec((g2, 1), lambda k: (0, 0)),
        ],
        out_specs=pl.BlockSpec((g2, bsz * n), lambda k: (0, 0)),
        out_shape=jax.ShapeDtypeStruct((g2, bsz * n), jnp.float32),
        scratch_shapes=[pltpu.VMEM((2, g1 * bsz, n), jnp.float32)],
    )(x0, L, w1b, b1b, W2, b2[:, None])

    h2 = h2g.reshape(g2, bsz, n).transpose(1, 2, 0).reshape(bsz, n * g2)

    nblk = 8
    chw = (n * g2) // nblk
    out = pl.pallas_call(
        _fc_kernel,
        grid=(nblk,),
        in_specs=[
            pl.BlockSpec((bsz, chw), lambda i: (0, i)),
            pl.BlockSpec((chw, d), lambda i: (i, 0)),
            pl.BlockSpec((1, d), lambda i: (0, 0)),
            pl.BlockSpec((d, c), lambda i: (0, 0)),
            pl.BlockSpec((1, c), lambda i: (0, 0)),
        ],
        out_specs=pl.BlockSpec((bsz, c), lambda i: (0, 0)),
        out_shape=jax.ShapeDtypeStruct((bsz, c), jnp.float32),
        scratch_shapes=[pltpu.VMEM((bsz, d), jnp.float32)],
    )(h2, fc1_W, fc1_b[None, :], fc2_W, fc2_b[None, :])
    return out
